# Initial kernel scaffold; baseline (speedup 1.0000x reference)
import jax
import jax.numpy as jnp
from jax.experimental import pallas as pl


def kernel(in_feat, edge_index_r1, edge_index_r2, edge_index_r3, W1, b1, W2, b2, W3, b3, W4, b4):
    raise NotImplementedError("write your pallas kernel here")


# speedup vs baseline: 3.7195x; 3.7195x over previous
import functools

import jax
import jax.numpy as jnp
from jax import lax
from jax.experimental import pallas as pl
from jax.experimental.pallas import tpu as pltpu
from jax.experimental.pallas import tpu_sc as plsc

N = 10000
E = 320000
F = 128
NC = 2
NS = 16
NW = NC * NS
EPW = E // NW
CH = 80
ROUNDS = EPW // CH
RPS = 624
RPS_LAST = N - (NS - 1) * RPS
BR = 1000
GRID = N // BR

_THETAS = ((3.0, -3.0, 0.75), (0.0, 3.0, -1.5), (0.0, 0.0, 0.75))

_mesh = plsc.VectorSubcoreMesh(core_axis_name="c", subcore_axis_name="s")


def _per_subcore_rows(sid, fn):
    @pl.when(sid < NS - 1)
    def _():
        fn(sid * RPS, RPS)

    @pl.when(sid == NS - 1)
    def _():
        fn((NS - 1) * RPS, RPS_LAST)


@functools.partial(
    pl.kernel,
    out_type=jax.ShapeDtypeStruct((NC, 3, N, F), jnp.float32),
    mesh=_mesh,
    scratch_types=[
        pltpu.VMEM((1, CH), jnp.int32),
        pltpu.VMEM((CH, F), jnp.float32),
        pltpu.VMEM_SHARED((N, F), jnp.float32),
    ],
)
def _deg_kernel(dst1, dst2, dst3, ones_hbm, zrows_hbm, out, didx, onesv, acc):
    cid = lax.axis_index("c")
    sid = lax.axis_index("s")
    wid = sid * NC + cid
    base = wid * EPW
    pltpu.sync_copy(ones_hbm, onesv)
    for r, dsth in enumerate((dst1, dst2, dst3)):
        _per_subcore_rows(sid, lambda s, n: pltpu.sync_copy(
            zrows_hbm.at[pl.ds(0, n)], acc.at[pl.ds(s, n)]))
        plsc.subcore_barrier()

        def body(k, _, dsth=dsth):
            pltpu.sync_copy(dsth.at[pl.ds(base + k * CH, CH)], didx.at[0])
            pltpu.sync_copy(onesv, acc.at[didx.at[0]], add=True)
            return 0

        lax.fori_loop(0, ROUNDS, body, 0)
        plsc.subcore_barrier()
        _per_subcore_rows(sid, lambda s, n, r=r: pltpu.sync_copy(
            acc.at[pl.ds(s, n)], out.at[cid, r, pl.ds(s, n)]))
        plsc.subcore_barrier()


@functools.partial(
    pl.kernel,
    out_type=jax.ShapeDtypeStruct((NC, N, F), jnp.float32),
    mesh=_mesh,
    scratch_types=[
        pltpu.VMEM((CH,), jnp.int32),
        pltpu.VMEM((1, CH), jnp.int32),
        pltpu.VMEM((CH, F), jnp.float32),
        pltpu.VMEM_SHARED((N, F), jnp.float32),
        pltpu.SemaphoreType.DMA,
    ],
)
def _lap_kernel(tmp_hbm, src_hbm, dst_hbm, zrows_hbm, out, sidx, didx, rows,
                acc, sem):
    cid = lax.axis_index("c")
    sid = lax.axis_index("s")
    wid = sid * NC + cid
    base = wid * EPW
    _per_subcore_rows(sid, lambda s, n: pltpu.sync_copy(
        zrows_hbm.at[pl.ds(0, n)], acc.at[pl.ds(s, n)]))
    plsc.subcore_barrier()

    def body(k, _):
        pltpu.sync_copy(src_hbm.at[pl.ds(base + k * CH, CH)], sidx)
        pltpu.sync_copy(dst_hbm.at[pl.ds(base + k * CH, CH)], didx.at[0])
        pltpu.async_copy(tmp_hbm.at[sidx], rows, sem).wait()
        pltpu.sync_copy(rows, acc.at[didx.at[0]], add=True)
        return 0

    lax.fori_loop(0, ROUNDS, body, 0)
    plsc.subcore_barrier()
    _per_subcore_rows(sid, lambda s, n: pltpu.sync_copy(
        acc.at[pl.ds(s, n)], out.at[cid, pl.ds(s, n)]))


def _leaky(x):
    return jnp.where(x >= 0, x, 0.01 * x)


def _row_spec(r=F):
    return pl.BlockSpec((BR, r), lambda i: (i, 0))


def _full(shape):
    return pl.BlockSpec(shape, lambda i: tuple(0 for _ in shape))


def _t1_body(x_ref, w1, b1, w2, b2, degp, h_ref, tmp_ref, dinv_ref):
    h = _leaky(jnp.dot(x_ref[...], w1[...],
                       preferred_element_type=jnp.float32) + b1[...])
    h = _leaky(jnp.dot(h, w2[...],
                       preferred_element_type=jnp.float32) + b2[...])
    h_ref[...] = h
    cols = []
    for r in range(3):
        dsum = degp[0, r, :, 0:1] + degp[1, r, :, 0:1]
        cols.append(lax.rsqrt(jnp.maximum(dsum, 1.0)))
    dinv_ref[...] = jnp.concatenate(cols, axis=1)
    tmp_ref[...] = h * cols[0]


def _t1(in_feat, w1t, b1, w2t, b2, degp):
    return pl.pallas_call(
        _t1_body,
        grid=(GRID,),
        in_specs=[_row_spec(), _full((F, F)), _full((1, F)), _full((F, F)),
                  _full((1, F)), pl.BlockSpec((NC, 3, BR, F),
                                              lambda i: (0, 0, i, 0))],
        out_specs=[_row_spec(), _row_spec(), _row_spec(3)],
        out_shape=[jax.ShapeDtypeStruct((N, F), jnp.float32),
                   jax.ShapeDtypeStruct((N, F), jnp.float32),
                   jax.ShapeDtypeStruct((N, 3), jnp.float32)],
    )(in_feat, w1t, b1, w2t, b2, degp)


def _upd_body(feat0, p, dcol, feat1_ref, tmp1_ref):
    d = dcol[...]
    feat1 = feat0[...] - (p[0] + p[1]) * d
    feat1_ref[...] = feat1
    tmp1_ref[...] = feat1 * d


def _t_update(feat0, p, dcol):
    return pl.pallas_call(
        _upd_body,
        grid=(GRID,),
        in_specs=[_row_spec(),
                  pl.BlockSpec((NC, BR, F), lambda i: (0, i, 0)),
                  _row_spec(1)],
        out_specs=[_row_spec(), _row_spec()],
        out_shape=[jax.ShapeDtypeStruct((N, F), jnp.float32),
                   jax.ShapeDtypeStruct((N, F), jnp.float32)],
    )(feat0, p, dcol)


def _dense_body(feat0, feat1, q, dcol, m0, m1, m2, b3, hsum_in, dnext,
                hnext_ref, tmpn_ref, hsum_ref):
    feat2 = feat1[...] - (q[0] + q[1]) * dcol[...]
    hnext = jnp.dot(feat0[...], m0[...], preferred_element_type=jnp.float32)
    hnext += jnp.dot(feat1[...], m1[...], preferred_element_type=jnp.float32)
    hnext += jnp.dot(feat2, m2[...], preferred_element_type=jnp.float32)
    hnext += b3[...]
    hnext_ref[...] = hnext
    hsum_ref[...] = hsum_in[...] + hnext
    tmpn_ref[...] = hnext * dnext[...]


def _t_dense(feat0, feat1, q, dcol, m0, m1, m2, b3, hsum_in, dnext):
    return pl.pallas_call(
        _dense_body,
        grid=(GRID,),
        in_specs=[_row_spec(), _row_spec(),
                  pl.BlockSpec((NC, BR, F), lambda i: (0, i, 0)),
                  _row_spec(1), _full((F, F)), _full((F, F)), _full((F, F)),
                  _full((1, F)), _row_spec(), _row_spec(1)],
        out_specs=[_row_spec(), _row_spec(), _row_spec()],
        out_shape=[jax.ShapeDtypeStruct((N, F), jnp.float32),
                   jax.ShapeDtypeStruct((N, F), jnp.float32),
                   jax.ShapeDtypeStruct((N, F), jnp.float32)],
    )(feat0, feat1, q, dcol, m0, m1, m2, b3, hsum_in, dnext)


def _final_body(hsum, w4t, b4, out_ref):
    out_ref[...] = jnp.dot(_leaky(hsum[...]), w4t[...],
                           preferred_element_type=jnp.float32) + b4[...]


def _t_final(hsum, w4t, b4):
    return pl.pallas_call(
        _final_body,
        grid=(GRID,),
        in_specs=[_row_spec(), _full((F, 2)), _full((1, 2))],
        out_specs=_row_spec(2),
        out_shape=jax.ShapeDtypeStruct((N, 2), jnp.float32),
    )(hsum, w4t, b4)


def kernel(in_feat, edge_index_r1, edge_index_r2, edge_index_r3,
           W1, b1, W2, b2, W3, b3, W4, b4):
    srcs = [jnp.asarray(ei[0], jnp.int32)
            for ei in (edge_index_r1, edge_index_r2, edge_index_r3)]
    dsts = [jnp.asarray(ei[1], jnp.int32)
            for ei in (edge_index_r1, edge_index_r2, edge_index_r3)]

    blocks = [W3[:, i * F:(i + 1) * F].T for i in range(3)]
    ms = [sum(_THETAS[i][j] * blocks[i] for i in range(3)) for j in range(3)]

    ones = jnp.ones((CH, F), jnp.float32)
    zrows = jnp.zeros((RPS_LAST, F), jnp.float32)

    degp = _deg_kernel(dsts[0], dsts[1], dsts[2], ones, zrows)

    feat0, tmp, dinv_all = _t1(in_feat, W1.T, b1[None, :], W2.T, b2[None, :],
                               degp)
    dcols = [dinv_all[:, r:r + 1] for r in range(3)]

    hsum = jnp.zeros((N, F), jnp.float32)
    for r in range(3):
        p = _lap_kernel(tmp, srcs[r], dsts[r], zrows)
        feat1, tmp1 = _t_update(feat0, p, dcols[r])
        q = _lap_kernel(tmp1, srcs[r], dsts[r], zrows)
        dnext = dcols[r + 1] if r < 2 else dcols[r]
        feat0, tmp, hsum = _t_dense(feat0, feat1, q, dcols[r], ms[0], ms[1],
                                    ms[2], b3[None, :], hsum, dnext)

    return _t_final(hsum, W4.T, b4[None, :])
